# baseline (speedup 1.0000x reference)
import functools

import jax
import jax.numpy as jnp
from jax import lax
from jax.experimental import pallas as pl
from jax.experimental.pallas import tpu as pltpu
from jax.experimental.pallas import tpu_sc as plsc

NUM_CORES = 2
NUM_SUBCORES = 16
NW = NUM_CORES * NUM_SUBCORES

BATCH, SEQ = 4096, 50
DIM = 728
ROWS = BATCH * SEQ
ROWS_PER_W = ROWS // NW
CHUNK = 64
NCHUNKS = ROWS_PER_W // CHUNK


@functools.partial(
    pl.kernel,
    out_type=jax.ShapeDtypeStruct((ROWS, DIM), jnp.float32),
    mesh=plsc.VectorSubcoreMesh(core_axis_name="c", subcore_axis_name="s"),
    scratch_types=[
        pltpu.VMEM((NCHUNKS, CHUNK), jnp.int32),
        pltpu.VMEM((CHUNK, DIM), jnp.float32),
        pltpu.VMEM((CHUNK, DIM), jnp.float32),
        pltpu.SemaphoreType.DMA,
        pltpu.SemaphoreType.DMA,
    ],
    compiler_params=pltpu.CompilerParams(use_tc_tiling_on_sc=False),
)
def _embedding_lookup(idx_hbm, table_hbm, out_hbm,
                      idx_v, buf0, buf1, gsem0, gsem1):
    wid = lax.axis_index("s") * NUM_CORES + lax.axis_index("c")
    base = wid * ROWS_PER_W
    pltpu.sync_copy(idx_hbm.at[wid], idx_v)

    bufs = (buf0, buf1)
    gsems = (gsem0, gsem1)

    def start_gather(c, t):
        pltpu.async_copy(table_hbm.at[idx_v.at[c]], bufs[t], gsems[t])

    start_gather(0, 0)
    start_gather(1, 1)

    @pl.loop(0, NCHUNKS, step=2)
    def _(jj):
        for t in range(2):
            c = jj + t
            pltpu.make_async_copy(
                table_hbm.at[idx_v.at[c]], bufs[t], gsems[t]).wait()
            pltpu.sync_copy(bufs[t], out_hbm.at[pl.ds(base + c * CHUNK, CHUNK)])

            @pl.when(c + 2 < NCHUNKS)
            def _():
                start_gather(c + 2, t)


def kernel(inputs, table):
    idx = jnp.asarray(inputs, jnp.int32).reshape(NW, NCHUNKS, CHUNK)
    out = _embedding_lookup(idx, table)
    return out.reshape(BATCH, SEQ, DIM)


# speedup vs baseline: 2.0026x; 2.0026x over previous
import functools

import jax
import jax.numpy as jnp
from jax import lax
from jax.experimental import pallas as pl
from jax.experimental.pallas import tpu as pltpu
from jax.experimental.pallas import tpu_sc as plsc

NUM_CORES = 2
NUM_SUBCORES = 16
NW = NUM_CORES * NUM_SUBCORES

BATCH, SEQ = 4096, 50
DIM = 728
ROWS = BATCH * SEQ
ROWS_PER_W = ROWS // NW
CHUNK = 64
NCHUNKS = ROWS_PER_W // CHUNK


@functools.partial(
    pl.kernel,
    out_type=jax.ShapeDtypeStruct((ROWS, DIM), jnp.float32),
    mesh=plsc.VectorSubcoreMesh(core_axis_name="c", subcore_axis_name="s"),
    scratch_types=[
        pltpu.VMEM((NCHUNKS, CHUNK), jnp.int32),
        pltpu.VMEM_SHARED((10, DIM), jnp.float32),
        pltpu.VMEM((CHUNK, DIM), jnp.float32),
        pltpu.VMEM((CHUNK, DIM), jnp.float32),
        pltpu.SemaphoreType.DMA,
        pltpu.SemaphoreType.DMA,
    ],
    compiler_params=pltpu.CompilerParams(use_tc_tiling_on_sc=False),
)
def _embedding_lookup(idx_hbm, table_hbm, out_hbm,
                      idx_v, table_v, buf0, buf1, gsem0, gsem1):
    wid = lax.axis_index("s") * NUM_CORES + lax.axis_index("c")
    base = wid * ROWS_PER_W
    pltpu.sync_copy(idx_hbm.at[wid], idx_v)

    @pl.when(lax.axis_index("s") == 0)
    def _():
        pltpu.sync_copy(table_hbm, table_v)

    plsc.subcore_barrier()

    bufs = (buf0, buf1)
    gsems = (gsem0, gsem1)

    def start_gather(c, t):
        pltpu.async_copy(table_v.at[idx_v.at[c]], bufs[t], gsems[t])

    start_gather(0, 0)
    start_gather(1, 1)

    @pl.loop(0, NCHUNKS, step=2)
    def _(jj):
        for t in range(2):
            c = jj + t
            pltpu.make_async_copy(
                table_v.at[idx_v.at[c]], bufs[t], gsems[t]).wait()
            pltpu.sync_copy(bufs[t], out_hbm.at[pl.ds(base + c * CHUNK, CHUNK)])

            @pl.when(c + 2 < NCHUNKS)
            def _():
                start_gather(c + 2, t)


def kernel(inputs, table):
    idx = jnp.asarray(inputs, jnp.int32).reshape(NW, NCHUNKS, CHUNK)
    out = _embedding_lookup(idx, table)
    return out.reshape(BATCH, SEQ, DIM)


# speedup vs baseline: 2.0065x; 1.0019x over previous
import functools

import jax
import jax.numpy as jnp
from jax import lax
from jax.experimental import pallas as pl
from jax.experimental.pallas import tpu as pltpu
from jax.experimental.pallas import tpu_sc as plsc

NUM_CORES = 2
NUM_SUBCORES = 16
NW = NUM_CORES * NUM_SUBCORES

BATCH, SEQ = 4096, 50
DIM = 728
VOCAB = 10
BATCH_PER_W = BATCH // NW


@functools.partial(
    pl.kernel,
    out_type=jax.ShapeDtypeStruct((BATCH, SEQ, DIM), jnp.float32),
    mesh=plsc.VectorSubcoreMesh(core_axis_name="c", subcore_axis_name="s"),
    scratch_types=[
        pltpu.VMEM((BATCH_PER_W, SEQ), jnp.int32),
        pltpu.VMEM_SHARED((VOCAB, DIM), jnp.float32),
        pltpu.VMEM((SEQ, DIM), jnp.float32),
        pltpu.VMEM((SEQ, DIM), jnp.float32),
        pltpu.SemaphoreType.DMA,
        pltpu.SemaphoreType.DMA,
    ],
    compiler_params=pltpu.CompilerParams(use_tc_tiling_on_sc=False),
)
def _embedding_lookup(idx_hbm, table_hbm, out_hbm,
                      idx_v, table_s, buf0, buf1, gsem0, gsem1):
    wid = lax.axis_index("s") * NUM_CORES + lax.axis_index("c")
    base = wid * BATCH_PER_W
    pltpu.sync_copy(idx_hbm.at[wid], idx_v)

    @pl.when(lax.axis_index("s") == 0)
    def _():
        pltpu.sync_copy(table_hbm, table_s)

    plsc.subcore_barrier()

    bufs = (buf0, buf1)
    gsems = (gsem0, gsem1)

    def start_gather(b, t):
        pltpu.async_copy(table_s.at[idx_v.at[b]], bufs[t], gsems[t])

    start_gather(0, 0)
    start_gather(1, 1)

    @pl.loop(0, BATCH_PER_W, step=2)
    def _(jj):
        for t in range(2):
            b = jj + t
            pltpu.make_async_copy(
                table_s.at[idx_v.at[b]], bufs[t], gsems[t]).wait()
            pltpu.sync_copy(bufs[t], out_hbm.at[base + b])

            @pl.when(b + 2 < BATCH_PER_W)
            def _():
                start_gather(b + 2, t)


def kernel(inputs, table):
    idx = jnp.asarray(inputs, jnp.int32).reshape(NW, BATCH_PER_W, SEQ)
    return _embedding_lookup(idx, table)
